# Initial kernel scaffold; baseline (speedup 1.0000x reference)
import jax
import jax.numpy as jnp
from jax.experimental import pallas as pl


def kernel(inputs):
    raise NotImplementedError("write your pallas kernel here")


# speedup vs baseline: 2.2423x; 2.2423x over previous
import jax
import jax.numpy as jnp
import numpy as np
from jax import lax
from jax.experimental import pallas as pl
from jax.experimental.pallas import tpu as pltpu
from jax.experimental.pallas import tpu_sc as plsc

KK = 8
B, H, W, C = 8, 224, 224, 192
HW = H * W
G = 49
NB = HW // G
L = 16
CW = 16
NROW = C // CW
RPB = G * NROW
NCAND = KK * G
NCAND_PAD = 400
NW = 32
TPW = (B * C) // NW
RB = 32

_NEG_INF = float("-inf")


def _blockstat_body(x_ref, mx_ref, mn_ref):
    x = x_ref[...]
    mx_ref[...] = jnp.max(x, axis=1)
    mn_ref[...] = jnp.min(x, axis=1)


def _transpose_body(mx_ref, mn_ref, mxt_ref, mnt_ref):
    mxt_ref[...] = jnp.transpose(mx_ref[...], (0, 2, 1))
    mnt_ref[...] = jnp.transpose(mn_ref[...], (0, 2, 1))


def _merge16(rv, ri, v, vi):
    sv, si = plsc.sort_key_val(v, vi, descending=True)
    keep = rv >= sv
    nv = jnp.where(keep, rv, sv)
    ni = jnp.where(keep, ri, si)
    srt = plsc.sort_key_val(nv, ni)
    return srt[0], srt[1]


def _scan_row(row_ref, nsteps, negate):
    iota = lax.iota(jnp.int32, L)

    def step(i, carry):
        rv, ri = carry
        v = row_ref[pl.ds(i * L, L)]
        if negate:
            v = -v
        return _merge16(rv, ri, v, iota + i * L)

    rv0 = jnp.full((L,), _NEG_INF, jnp.float32)
    ri0 = jnp.zeros((L,), jnp.int32)
    rv, ri = lax.fori_loop(0, nsteps, step, (rv0, ri0))
    return rv, ri


def _sc_body(x2d, mxt, mnt, pblk, poff, pval, outf,
             mrow_v, nrow_v, bb_v, idx_v, rows_v,
             pblk_v, poff_v, pval_v, out_v, sem):
    cid = lax.axis_index("c")
    sid = lax.axis_index("s")
    w = sid * 2 + cid
    b = w // 4
    c0 = 48 * (w % 4)
    iota = lax.iota(jnp.int32, L)

    pltpu.sync_copy(pblk, pblk_v)
    pltpu.sync_copy(poff, poff_v)
    pltpu.sync_copy(pval, pval_v)

    def task(tloc, _):
        c = c0 + tloc
        t = b * 192 + c
        crow = 3 * (w % 4) + lax.shift_right_logical(tloc, 4)
        clane = jnp.bitwise_and(tloc, 15)
        pltpu.sync_copy(mxt.at[pl.ds(t * NB, NB)], mrow_v)
        pltpu.sync_copy(mnt.at[pl.ds(t * NB, NB)], nrow_v)

        def side(row_ref, negate, lane_off):
            rv, ri = _scan_row(row_ref, NB // L, negate)
            rowbase = b * (HW * NROW) + crow
            bb_v[...] = ri * RPB + rowbase

            def build(i, _b):
                pb = pblk_v[pl.ds(i * L, L)]
                po = poff_v[pl.ds(i * L, L)]
                bv = plsc.load_gather(bb_v, [pb])
                idx_v[pl.ds(i * L, L)] = bv + po
                return 0

            lax.fori_loop(0, NCAND_PAD // L, build, 0)

            cps = []
            for k in range(3):
                cps.append(pltpu.async_copy(
                    x2d.at[idx_v.at[pl.ds(k * 128, 128)]],
                    rows_v.at[pl.ds(k * 128, 128)], sem))
            cps.append(pltpu.async_copy(
                x2d.at[idx_v.at[pl.ds(384, 16)]],
                rows_v.at[pl.ds(384, 16)], sem))
            for cp in cps:
                cp.wait()

            def cstep(i, rv2):
                cv = plsc.load_gather(rows_v, [iota + i * L,
                                               jnp.zeros((L,), jnp.int32) + clane])
                if negate:
                    cv = -cv
                cv = cv + pval_v[pl.ds(i * L, L)]
                nv, _nv2 = _merge16(rv2, rv2, cv, cv)
                return nv

            rv2 = lax.fori_loop(
                0, NCAND_PAD // L, cstep,
                jnp.full((L,), _NEG_INF, jnp.float32))

            best_desc = lax.rev(rv2, (0,))
            if negate:
                vals = -best_desc
            else:
                vals = best_desc
            plsc.store_scatter(out_v, [iota + (16 * tloc + lane_off)], vals,
                               mask=iota < 8)

        side(mrow_v, False, 0)
        side(nrow_v, True, 8)
        return 0

    lax.fori_loop(0, TPW, task, 0)
    pltpu.sync_copy(out_v, outf.at[pl.ds(w * (TPW * 16), TPW * 16)])


def _make_patterns():
    j = np.arange(NCAND_PAD)
    pblk = np.where(j < NCAND, 8 + j // G, 8).astype(np.int32)
    poff = np.where(j < NCAND, (j % G) * NROW, 0).astype(np.int32)
    pval = np.where(j < NCAND, 0.0, _NEG_INF).astype(np.float32)
    return jnp.asarray(pblk), jnp.asarray(poff), jnp.asarray(pval)


@jax.jit
def kernel(inputs):
    x3 = inputs.reshape(B * NB, G, C)

    mx, mn = pl.pallas_call(
        _blockstat_body,
        grid=(B * NB // RB,),
        in_specs=[pl.BlockSpec((RB, G, C), lambda i: (i, 0, 0))],
        out_specs=[pl.BlockSpec((RB, C), lambda i: (i, 0)),
                   pl.BlockSpec((RB, C), lambda i: (i, 0))],
        out_shape=[jax.ShapeDtypeStruct((B * NB, C), jnp.float32)] * 2,
    )(x3)

    mxt, mnt = pl.pallas_call(
        _transpose_body,
        grid=(B,),
        in_specs=[pl.BlockSpec((1, NB, C), lambda i: (i, 0, 0)),
                  pl.BlockSpec((1, NB, C), lambda i: (i, 0, 0))],
        out_specs=[pl.BlockSpec((1, C, NB), lambda i: (i, 0, 0)),
                   pl.BlockSpec((1, C, NB), lambda i: (i, 0, 0))],
        out_shape=[jax.ShapeDtypeStruct((B, C, NB), jnp.float32)] * 2,
    )(mx.reshape(B, NB, C), mn.reshape(B, NB, C))

    x2d = inputs.reshape(B * HW * NROW, CW)
    pblk, poff, pval = _make_patterns()

    mesh = plsc.VectorSubcoreMesh(core_axis_name="c", subcore_axis_name="s",
                                  num_cores=2, num_subcores=16)
    outf = pl.kernel(
        _sc_body,
        out_type=jax.ShapeDtypeStruct((B * C * 2 * KK,), jnp.float32),
        mesh=mesh,
        compiler_params=pltpu.CompilerParams(needs_layout_passes=False,
                                             use_tc_tiling_on_sc=False),
        scratch_types=[
            pltpu.VMEM((NB,), jnp.float32),
            pltpu.VMEM((NB,), jnp.float32),
            pltpu.VMEM((L,), jnp.int32),
            pltpu.VMEM((NCAND_PAD,), jnp.int32),
            pltpu.VMEM((NCAND_PAD, CW), jnp.float32),
            pltpu.VMEM((NCAND_PAD,), jnp.int32),
            pltpu.VMEM((NCAND_PAD,), jnp.int32),
            pltpu.VMEM((NCAND_PAD,), jnp.float32),
            pltpu.VMEM((TPW * 16,), jnp.float32),
            pltpu.SemaphoreType.DMA,
        ],
    )(x2d, mxt.reshape(B * C * NB), mnt.reshape(B * C * NB), pblk, poff, pval)

    return outf.reshape(B, 2 * KK * C)


# speedup vs baseline: 10.3851x; 4.6314x over previous
import jax
import jax.numpy as jnp
import numpy as np
from jax import lax
from jax.experimental import pallas as pl
from jax.experimental.pallas import tpu as pltpu
from jax.experimental.pallas import tpu_sc as plsc

KK = 8
B, H, W, C = 8, 224, 224, 192
HW = H * W
G = 32
NJ = H // G
NBLK = NJ * W
L = 16
NCAND = KK * G
NW = 32
TPW = (B * C) // NW
XCR = B * H * C
HSTRIDE = C * 8
ROWS16 = 2 * XCR * 8

_NEG_INF = float("-inf")


def _stage1_body(x_ref, mx_ref, mn_ref, xc_ref):
    x = x_ref[...]
    mx_ref[...] = jnp.max(x, axis=1, keepdims=True)
    mn_ref[...] = jnp.min(x, axis=1, keepdims=True)
    xc_ref[0] = x[0, :, :, 0:128].reshape(G * C, 128)
    xc_ref[1] = x[0, :, :, 96:224].reshape(G * C, 128)


def _transpose_body(mx_ref, mn_ref, mxt_ref, mnt_ref):
    mxt_ref[...] = jnp.transpose(mx_ref[...], (0, 2, 1, 3))
    mnt_ref[...] = jnp.transpose(mn_ref[...], (0, 2, 1, 3))


def _merge16(rv, ri, v, vi):
    sv, si = plsc.sort_key_val(v, vi, descending=True)
    keep = rv >= sv
    nv = jnp.where(keep, rv, sv)
    ni = jnp.where(keep, ri, si)
    srt = plsc.sort_key_val(nv, ni)
    return srt[0], srt[1]


def _scan_row(row_ref, nsteps, negate):
    iota = lax.iota(jnp.int32, L)

    def step(i, carry):
        rv, ri = carry
        v = row_ref[pl.ds(i * L, L)]
        if negate:
            v = -v
        return _merge16(rv, ri, v, iota + i * L)

    rv0 = jnp.full((L,), _NEG_INF, jnp.float32)
    ri0 = jnp.zeros((L,), jnp.int32)
    rv, ri = lax.fori_loop(0, nsteps, step, (rv0, ri0))
    return rv, ri


def _sc_body(x2d, mxt, mnt, boff, pblk, poff, outf,
             mrow_v, nrow_v, boff_v, bb_v, bl_v, idx_v, rows_v,
             pblk_v, poff_v, out_v, sem):
    cid = lax.axis_index("c")
    sid = lax.axis_index("s")
    w = sid * 2 + cid
    b = w // 4
    c0 = 48 * (w % 4)
    iota = lax.iota(jnp.int32, L)

    pltpu.sync_copy(boff, boff_v)
    pltpu.sync_copy(pblk, pblk_v)
    pltpu.sync_copy(poff, poff_v)

    def task(tloc, _):
        c = c0 + tloc
        t = b * C + c
        pltpu.sync_copy(mxt.at[pl.ds(t * NBLK, NBLK)], mrow_v)
        pltpu.sync_copy(mnt.at[pl.ds(t * NBLK, NBLK)], nrow_v)
        rowbase = b * (H * C * 8) + c * 8

        def side(row_ref, negate, lane_off):
            rv, ri = _scan_row(row_ref, NBLK // L, negate)
            bb_v[...] = plsc.load_gather(boff_v, [ri]) + rowbase
            bl_v[...] = jnp.bitwise_and(ri, L - 1)

            def build(i, _b):
                pb = pblk_v[pl.ds(i * L, L)]
                po = poff_v[pl.ds(i * L, L)]
                bv = plsc.load_gather(bb_v, [pb])
                idx_v[pl.ds(i * L, L)] = bv + po
                return 0

            lax.fori_loop(0, NCAND // L, build, 0)

            cp0 = pltpu.async_copy(x2d.at[idx_v.at[pl.ds(0, 128)]],
                                   rows_v.at[pl.ds(0, 128)], sem)
            cp1 = pltpu.async_copy(x2d.at[idx_v.at[pl.ds(128, 128)]],
                                   rows_v.at[pl.ds(128, 128)], sem)
            cp0.wait()
            cp1.wait()

            def cstep(i, rv2):
                pb = pblk_v[pl.ds(i * L, L)]
                lv = plsc.load_gather(bl_v, [pb])
                cv = plsc.load_gather(rows_v, [iota + i * L, lv])
                if negate:
                    cv = -cv
                nv, _nv2 = _merge16(rv2, rv2, cv, cv)
                return nv

            rv2 = lax.fori_loop(0, NCAND // L, cstep,
                                jnp.full((L,), _NEG_INF, jnp.float32))

            best_desc = lax.rev(rv2, (0,))
            if negate:
                vals = -best_desc
            else:
                vals = best_desc
            plsc.store_scatter(out_v, [iota + (16 * tloc + lane_off)], vals,
                               mask=iota < 8)

        side(mrow_v, False, 0)
        side(nrow_v, True, 8)
        return 0

    lax.fori_loop(0, TPW, task, 0)
    pltpu.sync_copy(out_v, outf.at[pl.ds(w * (TPW * 16), TPW * 16)])


def _make_tables():
    blk = np.arange(NBLK)
    jj, ww = blk // W, blk % W
    sel = (ww >= 128).astype(np.int64)
    wadj = ww - 96 * sel
    boff = (sel * (XCR * 8) + jj * G * HSTRIDE + wadj // L).astype(np.int32)
    j = np.arange(NCAND)
    pblk = (8 + j // G).astype(np.int32)
    poff = ((j % G) * HSTRIDE).astype(np.int32)
    return jnp.asarray(boff), jnp.asarray(pblk), jnp.asarray(poff)


@jax.jit
def kernel(inputs):
    xt = jnp.transpose(inputs, (0, 1, 3, 2))

    mx, mn, xcopy = pl.pallas_call(
        _stage1_body,
        grid=(B, NJ),
        in_specs=[pl.BlockSpec((1, G, C, W), lambda b, j: (b, j, 0, 0))],
        out_specs=[
            pl.BlockSpec((1, 1, C, W), lambda b, j: (b, j, 0, 0)),
            pl.BlockSpec((1, 1, C, W), lambda b, j: (b, j, 0, 0)),
            pl.BlockSpec((2, G * C, 128), lambda b, j: (0, b * NJ + j, 0)),
        ],
        out_shape=[
            jax.ShapeDtypeStruct((B, NJ, C, W), jnp.float32),
            jax.ShapeDtypeStruct((B, NJ, C, W), jnp.float32),
            jax.ShapeDtypeStruct((2, XCR, 128), jnp.float32),
        ],
    )(xt)

    mxt, mnt = pl.pallas_call(
        _transpose_body,
        grid=(B,),
        in_specs=[pl.BlockSpec((1, NJ, C, W), lambda b: (b, 0, 0, 0)),
                  pl.BlockSpec((1, NJ, C, W), lambda b: (b, 0, 0, 0))],
        out_specs=[pl.BlockSpec((1, C, NJ, W), lambda b: (b, 0, 0, 0)),
                   pl.BlockSpec((1, C, NJ, W), lambda b: (b, 0, 0, 0))],
        out_shape=[jax.ShapeDtypeStruct((B, C, NJ, W), jnp.float32)] * 2,
    )(mx, mn)

    x2d = xcopy.reshape(ROWS16, L)
    boff, pblk, poff = _make_tables()

    mesh = plsc.VectorSubcoreMesh(core_axis_name="c", subcore_axis_name="s",
                                  num_cores=2, num_subcores=16)
    outf = pl.kernel(
        _sc_body,
        out_type=jax.ShapeDtypeStruct((B * C * 2 * KK,), jnp.float32),
        mesh=mesh,
        compiler_params=pltpu.CompilerParams(needs_layout_passes=False,
                                             use_tc_tiling_on_sc=False),
        scratch_types=[
            pltpu.VMEM((NBLK,), jnp.float32),
            pltpu.VMEM((NBLK,), jnp.float32),
            pltpu.VMEM((NBLK,), jnp.int32),
            pltpu.VMEM((L,), jnp.int32),
            pltpu.VMEM((L,), jnp.int32),
            pltpu.VMEM((NCAND,), jnp.int32),
            pltpu.VMEM((NCAND, L), jnp.float32),
            pltpu.VMEM((NCAND,), jnp.int32),
            pltpu.VMEM((NCAND,), jnp.int32),
            pltpu.VMEM((TPW * 16,), jnp.float32),
            pltpu.SemaphoreType.DMA,
        ],
    )(x2d, mxt.reshape(B * C * NBLK), mnt.reshape(B * C * NBLK),
      boff, pblk, poff)

    return outf.reshape(B, 2 * KK * C)
